# baseline (speedup 1.0000x reference)
import functools

import jax
import jax.numpy as jnp
from jax import lax
from jax.experimental import pallas as pl
from jax.experimental.pallas import tpu as pltpu
from jax.experimental.pallas import tpu_sc as plsc

B = 16384
D = 32

_info = plsc.get_sparse_core_info()
_NC = _info.num_cores
_NS = _info.num_subcores
NW = _NC * _NS
BPW = B // NW
CHUNK = 128
NCHUNK = BPW // CHUNK

_mesh = plsc.VectorSubcoreMesh(core_axis_name="c", subcore_axis_name="s")


@functools.partial(
    pl.kernel,
    mesh=_mesh,
    compiler_params=pltpu.CompilerParams(use_tc_tiling_on_sc=False),
    out_type=[jax.ShapeDtypeStruct((B, D), jnp.float32)] * 4,
    scratch_types=[
        pltpu.VMEM((NCHUNK, CHUNK), jnp.int32),
        pltpu.VMEM((NCHUNK, CHUNK), jnp.int32),
        pltpu.VMEM((BPW, D), jnp.float32),
        pltpu.VMEM((BPW, D), jnp.float32),
        pltpu.VMEM((BPW, D), jnp.float32),
        pltpu.VMEM((BPW, D), jnp.float32),
        pltpu.SemaphoreType.DMA,
    ],
)
def _sc_gather4(uidx_hbm, iidx_hbm, ugt, igt, umt, imt,
                ug_o, ig_o, um_o, im_o,
                uix_v, iix_v, ug_v, ig_v, um_v, im_v, sem):
    wid = lax.axis_index("s") * _NC + lax.axis_index("c")
    base = wid * BPW
    pltpu.sync_copy(uidx_hbm.at[pl.ds(wid * NCHUNK, NCHUNK)], uix_v)
    pltpu.sync_copy(iidx_hbm.at[pl.ds(wid * NCHUNK, NCHUNK)], iix_v)
    copies = []
    for j in range(NCHUNK):
        rows = pl.ds(j * CHUNK, CHUNK)
        copies.append(pltpu.async_copy(ugt.at[uix_v.at[j]], ug_v.at[rows], sem))
        copies.append(pltpu.async_copy(igt.at[iix_v.at[j]], ig_v.at[rows], sem))
        copies.append(pltpu.async_copy(umt.at[uix_v.at[j]], um_v.at[rows], sem))
        copies.append(pltpu.async_copy(imt.at[iix_v.at[j]], im_v.at[rows], sem))
    for c in copies:
        c.wait()
    out_rows = pl.ds(base, BPW)
    pltpu.sync_copy(ug_v, ug_o.at[out_rows])
    pltpu.sync_copy(ig_v, ig_o.at[out_rows])
    pltpu.sync_copy(um_v, um_o.at[out_rows])
    pltpu.sync_copy(im_v, im_o.at[out_rows])


def _mlp_body(ug_ref, ig_ref, um_ref, im_ref,
              w1_ref, b1_ref, w2_ref, b2_ref, w3_ref, b3_ref,
              wg_ref, wh_ref, bo_ref, out_ref):
    um = um_ref[...]
    im = im_ref[...]
    w1u = w1_ref[:D, :]
    w1i = w1_ref[D:, :]
    h = jnp.dot(um, w1u, preferred_element_type=jnp.float32)
    h = h + jnp.dot(im, w1i, preferred_element_type=jnp.float32)
    h = jnp.maximum(h + b1_ref[...], 0.0)
    h = jnp.dot(h, w2_ref[...], preferred_element_type=jnp.float32)
    h = jnp.maximum(h + b2_ref[...], 0.0)
    h = jnp.dot(h, w3_ref[...], preferred_element_type=jnp.float32)
    h = jnp.maximum(h + b3_ref[...], 0.0)
    gmf = ug_ref[...] * ig_ref[...]
    z = (jnp.sum(gmf * wg_ref[...], axis=1)
         + jnp.sum(h * wh_ref[...], axis=1)
         + bo_ref[0])
    out_ref[...] = 1.0 / (1.0 + jnp.exp(-z))


def _mlp(ug, ig, um, im, w1t, b1, w2t, b2, w3t, b3, wg, wh, bout):
    bs = 2048
    grid = (B // bs,)
    emb_spec = pl.BlockSpec((bs, D), lambda i: (i, 0))

    def full(shape):
        return pl.BlockSpec(shape, lambda i: tuple(0 for _ in shape))

    return pl.pallas_call(
        _mlp_body,
        grid=grid,
        in_specs=[
            emb_spec, emb_spec, emb_spec, emb_spec,
            full((2 * D, 256)), full((1, 256)),
            full((256, 128)), full((1, 128)),
            full((128, 64)), full((1, 64)),
            full((1, D)), full((1, 64)),
            pl.BlockSpec(memory_space=pltpu.SMEM),
        ],
        out_specs=pl.BlockSpec((bs,), lambda i: (i,)),
        out_shape=jax.ShapeDtypeStruct((B,), jnp.float32),
    )(ug, ig, um, im, w1t, b1, w2t, b2, w3t, b3, wg, wh, bout)


def kernel(user_indices, item_indices, user_gmf_table, item_gmf_table,
           user_mlp_table, item_mlp_table, W1, b1, W2, b2, W3, b3, Wout, bout):
    uidx = user_indices.astype(jnp.int32).reshape(NW * NCHUNK, CHUNK)
    iidx = item_indices.astype(jnp.int32).reshape(NW * NCHUNK, CHUNK)
    ug, ig, um, im = _sc_gather4(uidx, iidx, user_gmf_table, item_gmf_table,
                                 user_mlp_table, item_mlp_table)
    w1t = W1.T
    w2t = W2.T
    w3t = W3.T
    wg = Wout[:, :D]
    wh = Wout[:, D:]
    return _mlp(ug, ig, um, im,
                w1t, b1.reshape(1, -1), w2t, b2.reshape(1, -1),
                w3t, b3.reshape(1, -1), wg, wh, bout)


# speedup vs baseline: 1.0006x; 1.0006x over previous
import functools

import jax
import jax.numpy as jnp
from jax import lax
from jax.experimental import pallas as pl
from jax.experimental.pallas import tpu as pltpu
from jax.experimental.pallas import tpu_sc as plsc

B = 16384
D = 32

_info = plsc.get_sparse_core_info()
_NC = _info.num_cores
_NS = _info.num_subcores
NW = _NC * _NS
BPW = B // NW
CHUNK = 128
NCHUNK = BPW // CHUNK

_mesh = plsc.VectorSubcoreMesh(core_axis_name="c", subcore_axis_name="s")


@functools.partial(
    pl.kernel,
    mesh=_mesh,
    compiler_params=pltpu.CompilerParams(use_tc_tiling_on_sc=False),
    out_type=[jax.ShapeDtypeStruct((B, D), jnp.float32)] * 4,
    scratch_types=[
        pltpu.VMEM((NCHUNK, CHUNK), jnp.int32),
        pltpu.VMEM((NCHUNK, CHUNK), jnp.int32),
        pltpu.VMEM((BPW, D), jnp.float32),
        pltpu.VMEM((BPW, D), jnp.float32),
        pltpu.VMEM((BPW, D), jnp.float32),
        pltpu.VMEM((BPW, D), jnp.float32),
        pltpu.SemaphoreType.DMA,
    ],
)
def _sc_gather4(uidx_hbm, iidx_hbm, ugt, igt, umt, imt,
                ug_o, ig_o, um_o, im_o,
                uix_v, iix_v, ug_v, ig_v, um_v, im_v, sem):
    wid = lax.axis_index("s") * _NC + lax.axis_index("c")
    base = wid * BPW
    pltpu.sync_copy(uidx_hbm.at[pl.ds(wid * NCHUNK, NCHUNK)], uix_v)
    pltpu.sync_copy(iidx_hbm.at[pl.ds(wid * NCHUNK, NCHUNK)], iix_v)
    copies = []
    for j in range(NCHUNK):
        rows = pl.ds(j * CHUNK, CHUNK)
        copies.append(pltpu.async_copy(ugt.at[uix_v.at[j]], ug_v.at[rows], sem))
        copies.append(pltpu.async_copy(igt.at[iix_v.at[j]], ig_v.at[rows], sem))
        copies.append(pltpu.async_copy(umt.at[uix_v.at[j]], um_v.at[rows], sem))
        copies.append(pltpu.async_copy(imt.at[iix_v.at[j]], im_v.at[rows], sem))
    for c in copies:
        c.wait()
    out_rows = pl.ds(base, BPW)
    pltpu.sync_copy(ug_v, ug_o.at[out_rows])
    pltpu.sync_copy(ig_v, ig_o.at[out_rows])
    pltpu.sync_copy(um_v, um_o.at[out_rows])
    pltpu.sync_copy(im_v, im_o.at[out_rows])


def _mlp_body(ug_ref, ig_ref, um_ref, im_ref,
              w1u_ref, w1i_ref, b1_ref, w2_ref, b2_ref, w3_ref, b3_ref,
              wg_ref, wh_ref, bo_ref, out_ref):
    um = um_ref[...]
    im = im_ref[...]
    h = jnp.dot(um, w1u_ref[...], preferred_element_type=jnp.float32)
    h = h + jnp.dot(im, w1i_ref[...], preferred_element_type=jnp.float32)
    h = jnp.maximum(h + b1_ref[...], 0.0)
    h = jnp.dot(h, w2_ref[...], preferred_element_type=jnp.float32)
    h = jnp.maximum(h + b2_ref[...], 0.0)
    h = jnp.dot(h, w3_ref[...], preferred_element_type=jnp.float32)
    h = jnp.maximum(h + b3_ref[...], 0.0)
    gmf = ug_ref[...] * ig_ref[...]
    z = (jnp.sum(gmf * wg_ref[...], axis=1)
         + jnp.sum(h * wh_ref[...], axis=1)
         + bo_ref[0])
    out_ref[...] = 1.0 / (1.0 + jnp.exp(-z))


def _mlp(ug, ig, um, im, w1u, w1i, b1, w2, b2, w3, b3, wg, wh, bout):
    bs = 2048
    grid = (B // bs,)
    emb_spec = pl.BlockSpec((bs, D), lambda i: (i, 0))

    def full(shape):
        return pl.BlockSpec(shape, lambda i: tuple(0 for _ in shape))

    return pl.pallas_call(
        _mlp_body,
        grid=grid,
        in_specs=[
            emb_spec, emb_spec, emb_spec, emb_spec,
            full((D, 256)), full((D, 256)), full((1, 256)),
            full((256, 128)), full((1, 128)),
            full((128, 64)), full((1, 64)),
            full((1, D)), full((1, 64)),
            pl.BlockSpec(memory_space=pltpu.SMEM),
        ],
        out_specs=pl.BlockSpec((bs,), lambda i: (i,)),
        out_shape=jax.ShapeDtypeStruct((B,), jnp.float32),
    )(ug, ig, um, im, w1u, w1i, b1, w2, b2, w3, b3, wg, wh, bout)


def kernel(user_indices, item_indices, user_gmf_table, item_gmf_table,
           user_mlp_table, item_mlp_table, W1, b1, W2, b2, W3, b3, Wout, bout):
    uidx = user_indices.astype(jnp.int32).reshape(NW * NCHUNK, CHUNK)
    iidx = item_indices.astype(jnp.int32).reshape(NW * NCHUNK, CHUNK)
    ug, ig, um, im = _sc_gather4(uidx, iidx, user_gmf_table, item_gmf_table,
                                 user_mlp_table, item_mlp_table)
    w1u = W1[:, :D].T
    w1i = W1[:, D:].T
    w2t = W2.T
    w3t = W3.T
    wg = Wout[:, :D]
    wh = Wout[:, D:]
    return _mlp(ug, ig, um, im,
                w1u, w1i, b1.reshape(1, -1), w2t, b2.reshape(1, -1),
                w3t, b3.reshape(1, -1), wg, wh, bout)
